# baseline (speedup 1.0000x reference)
import functools

import numpy as np

import jax
import jax.numpy as jnp
from jax import lax
from jax.experimental import pallas as pl
from jax.experimental.pallas import tpu as pltpu
from jax.experimental.pallas import tpu_sc as plsc

N = 50000
E = 800000
NMOL = 1000
EMB = 8
NSHIFT = 23
MSG = NSHIFT + EMB
NMOM = 12
W = 24
WACC = 24
WOUT = 32

NTILES = 32
CHUNK = 256
CHUNKS_PER_TILE = 99
EDGES_PER_TILE = CHUNK * CHUNKS_PER_TILE
E_PAD = EDGES_PER_TILE * NTILES
ROWS_PER_TILE = 3200
N_ACC = ROWS_PER_TILE * 16
ZROWS = 64
NXP = 12504
NSLICE = CHUNK // 128

NB = 2048
GRID = N_ACC // NB

_P = 128
_th = np.pi * (np.arange(_P) + 0.5) / _P
_d = (np.cos(_th) + 1.0) / 2.0 * 2.2 + 0.8
_G = np.exp(-(_d[:, None] - (0.8 + 0.1 * np.arange(NSHIFT))[None, :]) ** 2)
_C = np.cos(np.outer(np.arange(NMOM), _th))
_A = (2.0 / _P) * _C @ _G
_A[0] *= 0.5
_BT = np.zeros((WOUT, W), np.float32)
_BT[0:NSHIFT, 0:NMOM] = _A.T
_BT[NSHIFT:NSHIFT + EMB, NMOM:NMOM + EMB] = np.eye(EMB)


def _sc_aggregate(ea, ei, xp, embf):
    mesh = plsc.VectorSubcoreMesh(core_axis_name="c", subcore_axis_name="s",
                                  num_cores=2, num_subcores=16)

    @functools.partial(
        pl.kernel,
        out_type=jax.ShapeDtypeStruct((2, W, N_ACC), jnp.float32),
        mesh=mesh,
        compiler_params=pltpu.CompilerParams(needs_layout_passes=False,
                                             use_tc_tiling_on_sc=False),
        scratch_types=[
            pltpu.VMEM_SHARED((N_ACC, WACC), jnp.float32),
            pltpu.VMEM((CHUNK, WACC), jnp.float32),
            pltpu.VMEM((CHUNK, WACC), jnp.float32),
            pltpu.VMEM((CHUNK, WACC), jnp.float32),
            pltpu.VMEM((CHUNK,), jnp.float32),
            pltpu.VMEM((CHUNK,), jnp.float32),
            pltpu.VMEM((CHUNK,), jnp.float32),
            pltpu.VMEM((CHUNK,), jnp.int32),
            pltpu.VMEM((CHUNK,), jnp.int32),
            pltpu.VMEM((CHUNK,), jnp.int32),
            pltpu.VMEM((NSLICE, 128), jnp.int32),
            pltpu.VMEM((NSLICE, 128), jnp.int32),
            pltpu.VMEM((NSLICE, 128), jnp.int32),
            pltpu.VMEM((NXP,), jnp.int32),
            pltpu.VMEM((ZROWS, WACC), jnp.float32),
            pltpu.VMEM((128,), jnp.float32),
            pltpu.VMEM((128, WACC), jnp.float32),
            pltpu.VMEM((W, 128), jnp.float32),
            pltpu.SemaphoreType.DMA,
            pltpu.SemaphoreType.DMA,
        ],
    )
    def k(ea_hbm, ei_hbm, xp_hbm, emb_hbm, out_hbm,
          acc, rows0, rows1, rows2, d0, d1, d2, snk0, snk1, snk2,
          sidx0, sidx1, sidx2, xp_v, zbuf, emb_v, t24, t_t, sem_in, sem_sc):
        rows_b = (rows0, rows1, rows2)
        d_b = (d0, d1, d2)
        snk_b = (snk0, snk1, snk2)
        sidx_b = (sidx0, sidx1, sidx2)

        core = lax.axis_index("c")
        sub = lax.axis_index("s")
        wid = core * 16 + sub
        ebase = wid * EDGES_PER_TILE
        nchunks = jnp.where(wid == NTILES - 1, 56, CHUNKS_PER_TILE)
        zero16 = jnp.zeros((16,), jnp.float32)
        iota16 = lax.iota(jnp.int32, 16)

        pltpu.sync_copy(emb_hbm, emb_v.at[pl.ds(0, 96)])
        pltpu.sync_copy(xp_hbm, xp_v)

        def zb_body(r, _):
            zbuf[r, pl.ds(0, 16)] = zero16
            zbuf[r, pl.ds(8, 16)] = zero16
            return 0
        lax.fori_loop(0, ZROWS, zb_body, 0)
        tile_base = sub * ROWS_PER_TILE

        def zc_body(i, _):
            pltpu.sync_copy(zbuf, acc.at[pl.ds(tile_base + i * ZROWS, ZROWS)])
            return 0
        lax.fori_loop(0, ROWS_PER_TILE // ZROWS, zc_body, 0)

        for rows in rows_b:
            def zp_body(g, _):
                ridx = iota16 + g * 16
                for cpad in range(NMOM + EMB, WACC):
                    plsc.store_scatter(
                        rows, [ridx, jnp.full((16,), cpad, jnp.int32)], zero16)
                return 0
            lax.fori_loop(0, CHUNK // 16, zp_body, 0)

        plsc.subcore_barrier()

        def issue_in(c, s):
            base = ebase + c * CHUNK
            pltpu.async_copy(ea_hbm.at[0, pl.ds(base, CHUNK)], d_b[s], sem_in)
            pltpu.async_copy(ei_hbm.at[1, pl.ds(base, CHUNK)], snk_b[s], sem_in)
            for j in range(NSLICE):
                pltpu.async_copy(ei_hbm.at[0, pl.ds(base + j * 128, 128)],
                                 sidx_b[s].at[j], sem_in)

        def wait_in(s):
            pltpu.make_async_copy(ea_hbm.at[0, pl.ds(0, CHUNK)], d_b[s], sem_in).wait()
            pltpu.make_async_copy(ei_hbm.at[1, pl.ds(0, CHUNK)], snk_b[s], sem_in).wait()
            for j in range(NSLICE):
                pltpu.make_async_copy(ei_hbm.at[0, pl.ds(0, 128)],
                                      sidx_b[s].at[j], sem_in).wait()

        def issue_sc(s):
            for j in range(NSLICE):
                pltpu.async_copy(rows_b[s].at[pl.ds(j * 128, 128)],
                                 acc.at[sidx_b[s].at[j]], sem_sc, add=True)

        def wait_sc(s):
            for j in range(NSLICE):
                pltpu.make_async_copy(rows_b[s].at[pl.ds(j * 128, 128)],
                                      acc.at[sidx_b[s].at[j]], sem_sc).wait()

        def compute(c, s):
            rows, d, snkr = rows_b[s], d_b[s], snk_b[s]

            def grp_body(g, _):
                for h in range(2):
                    off = g * 32 + h * 16
                    dv = d[pl.ds(off, 16)]
                    ridx = iota16 + off
                    dt = (dv - 1.9) * (1.0 / 1.1)
                    t2 = dt + dt
                    plsc.store_scatter(rows, [ridx, jnp.full((16,), 0, jnp.int32)],
                                       jnp.full((16,), 1.0, jnp.float32))
                    plsc.store_scatter(rows, [ridx, jnp.full((16,), 1, jnp.int32)], dt)
                    tm2, tm1 = jnp.full((16,), 1.0, jnp.float32), dt
                    for m in range(2, NMOM):
                        tm = t2 * tm1 - tm2
                        plsc.store_scatter(rows, [ridx, jnp.full((16,), m, jnp.int32)], tm)
                        tm2, tm1 = tm1, tm
                    snkv = snkr[pl.ds(off, 16)]
                    word = plsc.load_gather(xp_v, [lax.shift_right_logical(snkv, 2)])
                    sh = lax.shift_left(jnp.bitwise_and(snkv, 3), 3)
                    cls = jnp.bitwise_and(lax.shift_right_logical(word, sh), 15)
                    base9 = lax.shift_left(cls, 3) + cls
                    for c8 in range(EMB):
                        v = plsc.load_gather(emb_v, [base9 + c8])
                        plsc.store_scatter(
                            rows, [ridx, jnp.full((16,), NMOM + c8, jnp.int32)], v)
                return 0
            lax.fori_loop(0, CHUNK // 32, grp_body, 0)

        issue_in(0, 0)

        def outer_body(o, _):
            for u in range(3):
                c = o * 3 + u
                s = u
                sn = (u + 1) % 3

                @pl.when(jnp.logical_and(c >= 2, c < nchunks))
                def _():
                    wait_sc(sn)

                @pl.when(c + 1 < nchunks)
                def _():
                    issue_in(c + 1, sn)

                @pl.when(c < nchunks)
                def _():
                    wait_in(s)
                    compute(c, s)
                    issue_sc(s)
            return 0
        lax.fori_loop(0, CHUNKS_PER_TILE // 3, outer_body, 0)
        wait_sc(1)

        @pl.when(wid != NTILES - 1)
        def _():
            wait_sc(2)

        @pl.when(wid == NTILES - 1)
        def _():
            wait_sc(0)

        plsc.subcore_barrier()

        def t_body(i, _):
            nb = tile_base + i * 128
            pltpu.sync_copy(acc.at[pl.ds(nb, 128)], t24)
            for f in range(W):
                fvec = jnp.full((16,), f, jnp.int32)
                for j2 in range(8):
                    v = plsc.load_gather(t24, [iota16 + j2 * 16, fvec])
                    t_t[f, pl.ds(j2 * 16, 16)] = v
            pltpu.sync_copy(t_t, out_hbm.at[core].at[:, pl.ds(nb, 128)])
            return 0
        lax.fori_loop(0, ROWS_PER_TILE // 128, t_body, 0)

    return k(ea, ei, xp, embf)


def _b1_body(parts_ref, bt_ref, stats_ref):
    i = pl.program_id(0)
    m = parts_ref[0] + parts_ref[1]
    feat = jnp.dot(bt_ref[...], m, preferred_element_type=jnp.float32)
    gidx = lax.broadcasted_iota(jnp.int32, (1, NB), 1) + i * NB
    feat = feat * (gidx < N).astype(jnp.float32)

    @pl.when(i == 0)
    def _():
        stats_ref[...] = jnp.zeros_like(stats_ref)

    stats_ref[:, 0:1] += jnp.sum(feat, axis=1, keepdims=True)
    stats_ref[:, 1:2] += jnp.sum(feat * feat, axis=1, keepdims=True)


def _b2_body(parts_ref, x_ref, bt_ref, stats_ref, g_ref, b_ref,
             w1_ref, b1_ref, w2_ref, b2_ref, w3_ref, b3_ref, w4_ref, b4_ref,
             embt_ref, xall_ref, stats2_ref):
    i = pl.program_id(0)
    mean = stats_ref[:, 0:1] * (1.0 / N)
    var = stats_ref[:, 1:2] * (1.0 / N) - mean * mean
    scale = g_ref[...] * lax.rsqrt(var + 1e-5)
    shift = b_ref[...] - mean * scale

    m = parts_ref[0] + parts_ref[1]
    feat = jnp.dot(bt_ref[...], m, preferred_element_type=jnp.float32)
    h = feat * scale + shift
    bf = jnp.bfloat16
    h = jnp.maximum(jnp.dot(w1_ref[...], h.astype(bf), preferred_element_type=jnp.float32) + b1_ref[...], 0.0)
    h = jnp.maximum(jnp.dot(w2_ref[...], h.astype(bf), preferred_element_type=jnp.float32) + b2_ref[...], 0.0)
    h = jnp.maximum(jnp.dot(w3_ref[...], h.astype(bf), preferred_element_type=jnp.float32) + b3_ref[...], 0.0)
    h = jnp.dot(w4_ref[...], h.astype(bf), preferred_element_type=jnp.float32) + b4_ref[...]

    oh = (lax.broadcasted_iota(jnp.int32, (16, NB), 0) == x_ref[...]).astype(jnp.float32)
    x0 = jnp.dot(embt_ref[...], oh, preferred_element_type=jnp.float32)
    x1 = x0 + 0.1 * h
    xall = jnp.concatenate([x0, x1], axis=0)
    xall_ref[...] = xall

    @pl.when(i == 0)
    def _():
        stats2_ref[...] = jnp.zeros_like(stats2_ref)

    gidx = lax.broadcasted_iota(jnp.int32, (1, NB), 1) + i * NB
    xm = xall * (gidx < N).astype(jnp.float32)
    stats2_ref[:, 0:1] += jnp.sum(xm, axis=1, keepdims=True)
    stats2_ref[:, 1:2] += jnp.sum(xm * xm, axis=1, keepdims=True)


def _b3_body(xall_ref, batch_ref, stats2_ref, g_ref, b_ref,
             w1_ref, b1_ref, w2_ref, b2_ref, w3_ref, b3_ref, w4_ref, b4_ref,
             y_ref):
    i = pl.program_id(0)
    mean = stats2_ref[:, 0:1] * (1.0 / N)
    var = stats2_ref[:, 1:2] * (1.0 / N) - mean * mean
    scale = g_ref[...] * lax.rsqrt(var + 1e-5)
    shift = b_ref[...] - mean * scale

    bf = jnp.bfloat16
    h = xall_ref[...] * scale + shift
    h = jnp.maximum(jnp.dot(w1_ref[...], h.astype(bf), preferred_element_type=jnp.float32) + b1_ref[...], 0.0)
    h = jnp.maximum(jnp.dot(w2_ref[...], h.astype(bf), preferred_element_type=jnp.float32) + b2_ref[...], 0.0)
    h = jnp.maximum(jnp.dot(w3_ref[...], h.astype(bf), preferred_element_type=jnp.float32) + b3_ref[...], 0.0)
    y_i = jnp.dot(w4_ref[...], h.astype(bf), preferred_element_type=jnp.float32) + b4_ref[...]

    bv = batch_ref[...]
    ohh = (lax.broadcasted_iota(jnp.int32, (32, NB), 0)
           == lax.shift_right_logical(bv, 5)).astype(jnp.float32)
    ohl = (lax.broadcasted_iota(jnp.int32, (32, NB), 0)
           == jnp.bitwise_and(bv, 31)).astype(jnp.float32)
    c = lax.dot_general(ohh * y_i, ohl, (((1,), (1,)), ((), ())),
                        preferred_element_type=jnp.float32)

    @pl.when(i == 0)
    def _():
        y_ref[...] = jnp.zeros_like(y_ref)

    y_ref[...] += c


def kernel(x, edge_index, edge_attr, batch, emb, up_bn_g, up_bn_b, up_w1, up_b1,
           up_w2, up_b2, up_w3, up_b3, up_w4, up_b4, ro_bn_g, ro_bn_b, ro_w1,
           ro_b1, ro_w2, ro_b2, ro_w3, ro_b3, ro_w4, ro_b4):
    f32 = jnp.float32
    i32 = jnp.int32
    ea = edge_attr.reshape(1, E)
    xu8 = jnp.pad(x.astype(jnp.uint8), (0, 4 * NXP - N))
    xp = lax.bitcast_convert_type(xu8.reshape(NXP, 4), i32)
    emb9 = jnp.zeros((10, 9), f32).at[:, :EMB].set(emb)
    embf = jnp.pad(emb9.reshape(90), (0, 6))

    parts = _sc_aggregate(ea, edge_index.astype(i32), xp, embf)
    bt = jnp.asarray(_BT)

    x_row = jnp.pad(x.astype(i32), (0, N_ACC - N),
                    constant_values=10).reshape(1, N_ACC)
    batch_row = jnp.pad(batch.astype(i32), (0, N_ACC - N),
                        constant_values=1023).reshape(1, N_ACC)

    stats = pl.pallas_call(
        _b1_body,
        grid=(GRID,),
        in_specs=[pl.BlockSpec((2, W, NB), lambda i: (0, 0, i)),
                  pl.BlockSpec((WOUT, W), lambda i: (0, 0))],
        out_specs=pl.BlockSpec((WOUT, 128), lambda i: (0, 0)),
        out_shape=jax.ShapeDtypeStruct((WOUT, 128), f32),
    )(parts, bt)

    bf = jnp.bfloat16
    w1p = jnp.zeros((200, WOUT), f32).at[:, :MSG].set(up_w1).astype(bf)
    gp = jnp.zeros((WOUT, 1), f32).at[:MSG, 0].set(up_bn_g)
    bp = jnp.zeros((WOUT, 1), f32).at[:MSG, 0].set(up_bn_b)
    embt = jnp.zeros((EMB, 16), f32).at[:, :10].set(emb.T)

    const = lambda shape: pl.BlockSpec(shape, lambda i: tuple(0 for _ in shape))
    xall, stats2 = pl.pallas_call(
        _b2_body,
        grid=(GRID,),
        in_specs=[
            pl.BlockSpec((2, W, NB), lambda i: (0, 0, i)),
            pl.BlockSpec((1, NB), lambda i: (0, i)),
            const((WOUT, W)),
            const((WOUT, 128)), const((WOUT, 1)), const((WOUT, 1)),
            const((200, WOUT)), const((200, 1)),
            const((200, 200)), const((200, 1)),
            const((200, 200)), const((200, 1)),
            const((EMB, 200)), const((EMB, 1)),
            const((EMB, 16)),
        ],
        out_specs=[
            pl.BlockSpec((2 * EMB, NB), lambda i: (0, i)),
            pl.BlockSpec((2 * EMB, 128), lambda i: (0, 0)),
        ],
        out_shape=[
            jax.ShapeDtypeStruct((2 * EMB, N_ACC), f32),
            jax.ShapeDtypeStruct((2 * EMB, 128), f32),
        ],
    )(parts, x_row, bt, stats, gp, bp,
      w1p, up_b1.reshape(200, 1), up_w2.astype(bf), up_b2.reshape(200, 1),
      up_w3.astype(bf), up_b3.reshape(200, 1), up_w4.astype(bf),
      up_b4.reshape(EMB, 1), embt)

    y32 = pl.pallas_call(
        _b3_body,
        grid=(GRID,),
        in_specs=[
            pl.BlockSpec((2 * EMB, NB), lambda i: (0, i)),
            pl.BlockSpec((1, NB), lambda i: (0, i)),
            const((2 * EMB, 128)), const((2 * EMB, 1)), const((2 * EMB, 1)),
            const((200, 2 * EMB)), const((200, 1)),
            const((200, 200)), const((200, 1)),
            const((200, 200)), const((200, 1)),
            const((1, 200)), const((1, 1)),
        ],
        out_specs=pl.BlockSpec((32, 32), lambda i: (0, 0)),
        out_shape=jax.ShapeDtypeStruct((32, 32), f32),
    )(xall, batch_row, stats2,
      ro_bn_g.reshape(2 * EMB, 1), ro_bn_b.reshape(2 * EMB, 1),
      ro_w1.astype(bf), ro_b1.reshape(200, 1), ro_w2.astype(bf),
      ro_b2.reshape(200, 1), ro_w3.astype(bf), ro_b3.reshape(200, 1),
      ro_w4.astype(bf), ro_b4.reshape(1, 1))

    return y32.reshape(1024, 1)[:NMOL]


# speedup vs baseline: 1.0929x; 1.0929x over previous
import functools

import numpy as np

import jax
import jax.numpy as jnp
from jax import lax
from jax.experimental import pallas as pl
from jax.experimental.pallas import tpu as pltpu
from jax.experimental.pallas import tpu_sc as plsc

N = 50000
E = 800000
NMOL = 1000
EMB = 8
NSHIFT = 23
MSG = NSHIFT + EMB
NMOM = 12
W = 24
WACC = 24
WOUT = 32

NTILES = 32
CHUNK = 256
CHUNKS_PER_TILE = 99
EDGES_PER_TILE = CHUNK * CHUNKS_PER_TILE
E_PAD = EDGES_PER_TILE * NTILES
ROWS_PER_TILE = 3200
N_ACC = ROWS_PER_TILE * 16
ZROWS = 64
NXP = 12504
NSLICE = CHUNK // 128

NB = 6400
GRID = N_ACC // NB

_P = 128
_th = np.pi * (np.arange(_P) + 0.5) / _P
_d = (np.cos(_th) + 1.0) / 2.0 * 2.2 + 0.8
_G = np.exp(-(_d[:, None] - (0.8 + 0.1 * np.arange(NSHIFT))[None, :]) ** 2)
_C = np.cos(np.outer(np.arange(NMOM), _th))
_A = (2.0 / _P) * _C @ _G
_A[0] *= 0.5
_BT = np.zeros((WOUT, W), np.float32)
_BT[0:NSHIFT, 0:NMOM] = _A.T
_BT[NSHIFT:NSHIFT + EMB, NMOM:NMOM + EMB] = np.eye(EMB)


def _sc_aggregate(ea, ei, xp, embf):
    mesh = plsc.VectorSubcoreMesh(core_axis_name="c", subcore_axis_name="s",
                                  num_cores=2, num_subcores=16)

    @functools.partial(
        pl.kernel,
        out_type=jax.ShapeDtypeStruct((2, W, N_ACC), jnp.float32),
        mesh=mesh,
        compiler_params=pltpu.CompilerParams(needs_layout_passes=False,
                                             use_tc_tiling_on_sc=False),
        scratch_types=[
            pltpu.VMEM_SHARED((N_ACC, WACC), jnp.float32),
            pltpu.VMEM((CHUNK, WACC), jnp.float32),
            pltpu.VMEM((CHUNK, WACC), jnp.float32),
            pltpu.VMEM((CHUNK, WACC), jnp.float32),
            pltpu.VMEM((CHUNK,), jnp.float32),
            pltpu.VMEM((CHUNK,), jnp.float32),
            pltpu.VMEM((CHUNK,), jnp.float32),
            pltpu.VMEM((CHUNK,), jnp.int32),
            pltpu.VMEM((CHUNK,), jnp.int32),
            pltpu.VMEM((CHUNK,), jnp.int32),
            pltpu.VMEM((NSLICE, 128), jnp.int32),
            pltpu.VMEM((NSLICE, 128), jnp.int32),
            pltpu.VMEM((NSLICE, 128), jnp.int32),
            pltpu.VMEM((NXP,), jnp.int32),
            pltpu.VMEM((ZROWS, WACC), jnp.float32),
            pltpu.VMEM((128,), jnp.float32),
            pltpu.VMEM((128, WACC), jnp.float32),
            pltpu.VMEM((W, 128), jnp.float32),
            pltpu.SemaphoreType.DMA,
            pltpu.SemaphoreType.DMA,
        ],
    )
    def k(ea_hbm, ei_hbm, xp_hbm, emb_hbm, out_hbm,
          acc, rows0, rows1, rows2, d0, d1, d2, snk0, snk1, snk2,
          sidx0, sidx1, sidx2, xp_v, zbuf, emb_v, t24, t_t, sem_in, sem_sc):
        rows_b = (rows0, rows1, rows2)
        d_b = (d0, d1, d2)
        snk_b = (snk0, snk1, snk2)
        sidx_b = (sidx0, sidx1, sidx2)

        core = lax.axis_index("c")
        sub = lax.axis_index("s")
        wid = core * 16 + sub
        ebase = wid * EDGES_PER_TILE
        nchunks = jnp.where(wid == NTILES - 1, 56, CHUNKS_PER_TILE)
        zero16 = jnp.zeros((16,), jnp.float32)
        iota16 = lax.iota(jnp.int32, 16)

        pltpu.sync_copy(emb_hbm, emb_v.at[pl.ds(0, 96)])
        pltpu.sync_copy(xp_hbm, xp_v)

        def zb_body(r, _):
            zbuf[r, pl.ds(0, 16)] = zero16
            zbuf[r, pl.ds(8, 16)] = zero16
            return 0
        lax.fori_loop(0, ZROWS, zb_body, 0)
        tile_base = sub * ROWS_PER_TILE

        def zc_body(i, _):
            pltpu.sync_copy(zbuf, acc.at[pl.ds(tile_base + i * ZROWS, ZROWS)])
            return 0
        lax.fori_loop(0, ROWS_PER_TILE // ZROWS, zc_body, 0)

        for rows in rows_b:
            def zp_body(g, _):
                ridx = iota16 + g * 16
                for cpad in range(NMOM + EMB, WACC):
                    plsc.store_scatter(
                        rows, [ridx, jnp.full((16,), cpad, jnp.int32)], zero16)
                return 0
            lax.fori_loop(0, CHUNK // 16, zp_body, 0)

        plsc.subcore_barrier()

        def issue_in(c, s):
            base = ebase + c * CHUNK
            pltpu.async_copy(ea_hbm.at[pl.ds(base, CHUNK)], d_b[s], sem_in)
            pltpu.async_copy(ei_hbm.at[1, pl.ds(base, CHUNK)], snk_b[s], sem_in)
            for j in range(NSLICE):
                pltpu.async_copy(ei_hbm.at[0, pl.ds(base + j * 128, 128)],
                                 sidx_b[s].at[j], sem_in)

        def wait_in(s):
            pltpu.make_async_copy(ea_hbm.at[pl.ds(0, CHUNK)], d_b[s], sem_in).wait()
            pltpu.make_async_copy(ei_hbm.at[1, pl.ds(0, CHUNK)], snk_b[s], sem_in).wait()
            for j in range(NSLICE):
                pltpu.make_async_copy(ei_hbm.at[0, pl.ds(0, 128)],
                                      sidx_b[s].at[j], sem_in).wait()

        def issue_sc(s):
            for j in range(NSLICE):
                pltpu.async_copy(rows_b[s].at[pl.ds(j * 128, 128)],
                                 acc.at[sidx_b[s].at[j]], sem_sc, add=True)

        def wait_sc(s):
            for j in range(NSLICE):
                pltpu.make_async_copy(rows_b[s].at[pl.ds(j * 128, 128)],
                                      acc.at[sidx_b[s].at[j]], sem_sc).wait()

        def compute(c, s):
            rows, d, snkr = rows_b[s], d_b[s], snk_b[s]

            def grp_body(g, _):
                for h in range(2):
                    off = g * 32 + h * 16
                    dv = d[pl.ds(off, 16)]
                    ridx = iota16 + off
                    dt = (dv - 1.9) * (1.0 / 1.1)
                    t2 = dt + dt
                    plsc.store_scatter(rows, [ridx, jnp.full((16,), 0, jnp.int32)],
                                       jnp.full((16,), 1.0, jnp.float32))
                    plsc.store_scatter(rows, [ridx, jnp.full((16,), 1, jnp.int32)], dt)
                    tm2, tm1 = jnp.full((16,), 1.0, jnp.float32), dt
                    for m in range(2, NMOM):
                        tm = t2 * tm1 - tm2
                        plsc.store_scatter(rows, [ridx, jnp.full((16,), m, jnp.int32)], tm)
                        tm2, tm1 = tm1, tm
                    snkv = snkr[pl.ds(off, 16)]
                    word = plsc.load_gather(xp_v, [lax.shift_right_logical(snkv, 2)])
                    sh = lax.shift_left(jnp.bitwise_and(snkv, 3), 3)
                    cls = jnp.bitwise_and(lax.shift_right_logical(word, sh), 15)
                    base9 = lax.shift_left(cls, 3) + cls
                    for c8 in range(EMB):
                        v = plsc.load_gather(emb_v, [base9 + c8])
                        plsc.store_scatter(
                            rows, [ridx, jnp.full((16,), NMOM + c8, jnp.int32)], v)
                return 0
            lax.fori_loop(0, CHUNK // 32, grp_body, 0)

        issue_in(0, 0)

        def outer_body(o, _):
            for u in range(3):
                c = o * 3 + u
                s = u
                sn = (u + 1) % 3

                @pl.when(jnp.logical_and(c >= 2, c < nchunks))
                def _():
                    wait_sc(sn)

                @pl.when(c + 1 < nchunks)
                def _():
                    issue_in(c + 1, sn)

                @pl.when(c < nchunks)
                def _():
                    wait_in(s)
                    compute(c, s)
                    issue_sc(s)
            return 0
        lax.fori_loop(0, CHUNKS_PER_TILE // 3, outer_body, 0)
        wait_sc(1)

        @pl.when(wid != NTILES - 1)
        def _():
            wait_sc(2)

        @pl.when(wid == NTILES - 1)
        def _():
            wait_sc(0)

        plsc.subcore_barrier()

        def t_body(i, _):
            nb = tile_base + i * 128
            pltpu.sync_copy(acc.at[pl.ds(nb, 128)], t24)
            for f in range(W):
                fvec = jnp.full((16,), f, jnp.int32)
                for j2 in range(8):
                    v = plsc.load_gather(t24, [iota16 + j2 * 16, fvec])
                    t_t[f, pl.ds(j2 * 16, 16)] = v
            pltpu.sync_copy(t_t, out_hbm.at[core].at[:, pl.ds(nb, 128)])
            return 0
        lax.fori_loop(0, ROWS_PER_TILE // 128, t_body, 0)

    return k(ea, ei, xp, embf)


def _b1_body(parts_ref, bt_ref, stats_ref):
    i = pl.program_id(0)
    m = parts_ref[0] + parts_ref[1]
    feat = jnp.dot(bt_ref[...], m, preferred_element_type=jnp.float32)
    gidx = lax.broadcasted_iota(jnp.int32, (1, NB), 1) + i * NB
    feat = feat * (gidx < N).astype(jnp.float32)

    @pl.when(i == 0)
    def _():
        stats_ref[...] = jnp.zeros_like(stats_ref)

    stats_ref[:, 0:1] += jnp.sum(feat, axis=1, keepdims=True)
    stats_ref[:, 1:2] += jnp.sum(feat * feat, axis=1, keepdims=True)


def _b2_body(parts_ref, x_ref, bt_ref, stats_ref, g_ref, b_ref,
             w1_ref, b1_ref, w2_ref, b2_ref, w3_ref, b3_ref, w4_ref, b4_ref,
             embt_ref, xall_ref, stats2_ref):
    i = pl.program_id(0)
    mean = stats_ref[:, 0:1] * (1.0 / N)
    var = stats_ref[:, 1:2] * (1.0 / N) - mean * mean
    scale = g_ref[...] * lax.rsqrt(var + 1e-5)
    shift = b_ref[...] - mean * scale

    m = parts_ref[0] + parts_ref[1]
    feat = jnp.dot(bt_ref[...], m, preferred_element_type=jnp.float32)
    h = feat * scale + shift
    bf = jnp.bfloat16
    h = jnp.maximum(jnp.dot(w1_ref[...], h.astype(bf), preferred_element_type=jnp.float32) + b1_ref[...], 0.0)
    h = jnp.maximum(jnp.dot(w2_ref[...], h.astype(bf), preferred_element_type=jnp.float32) + b2_ref[...], 0.0)
    h = jnp.maximum(jnp.dot(w3_ref[...], h.astype(bf), preferred_element_type=jnp.float32) + b3_ref[...], 0.0)
    h = jnp.dot(w4_ref[...], h.astype(bf), preferred_element_type=jnp.float32) + b4_ref[...]

    oh = (lax.broadcasted_iota(jnp.int32, (16, NB), 0) == x_ref[...]).astype(jnp.float32)
    x0 = jnp.dot(embt_ref[...], oh, preferred_element_type=jnp.float32)
    x1 = x0 + 0.1 * h
    xall = jnp.concatenate([x0, x1], axis=0)
    xall_ref[...] = xall

    @pl.when(i == 0)
    def _():
        stats2_ref[...] = jnp.zeros_like(stats2_ref)

    gidx = lax.broadcasted_iota(jnp.int32, (1, NB), 1) + i * NB
    xm = xall * (gidx < N).astype(jnp.float32)
    stats2_ref[:, 0:1] += jnp.sum(xm, axis=1, keepdims=True)
    stats2_ref[:, 1:2] += jnp.sum(xm * xm, axis=1, keepdims=True)


def _b3_body(xall_ref, batch_ref, stats2_ref, g_ref, b_ref,
             w1_ref, b1_ref, w2_ref, b2_ref, w3_ref, b3_ref, w4_ref, b4_ref,
             y_ref):
    i = pl.program_id(0)
    mean = stats2_ref[:, 0:1] * (1.0 / N)
    var = stats2_ref[:, 1:2] * (1.0 / N) - mean * mean
    scale = g_ref[...] * lax.rsqrt(var + 1e-5)
    shift = b_ref[...] - mean * scale

    bf = jnp.bfloat16
    h = xall_ref[...] * scale + shift
    h = jnp.maximum(jnp.dot(w1_ref[...], h.astype(bf), preferred_element_type=jnp.float32) + b1_ref[...], 0.0)
    h = jnp.maximum(jnp.dot(w2_ref[...], h.astype(bf), preferred_element_type=jnp.float32) + b2_ref[...], 0.0)
    h = jnp.maximum(jnp.dot(w3_ref[...], h.astype(bf), preferred_element_type=jnp.float32) + b3_ref[...], 0.0)
    y_i = jnp.dot(w4_ref[...], h.astype(bf), preferred_element_type=jnp.float32) + b4_ref[...]

    bv = batch_ref[...]
    ohh = (lax.broadcasted_iota(jnp.int32, (32, NB), 0)
           == lax.shift_right_logical(bv, 5)).astype(jnp.float32)
    ohl = (lax.broadcasted_iota(jnp.int32, (32, NB), 0)
           == jnp.bitwise_and(bv, 31)).astype(jnp.float32)
    c = lax.dot_general(ohh * y_i, ohl, (((1,), (1,)), ((), ())),
                        preferred_element_type=jnp.float32)

    @pl.when(i == 0)
    def _():
        y_ref[...] = jnp.zeros_like(y_ref)

    y_ref[...] += c


def kernel(x, edge_index, edge_attr, batch, emb, up_bn_g, up_bn_b, up_w1, up_b1,
           up_w2, up_b2, up_w3, up_b3, up_w4, up_b4, ro_bn_g, ro_bn_b, ro_w1,
           ro_b1, ro_w2, ro_b2, ro_w3, ro_b3, ro_w4, ro_b4):
    f32 = jnp.float32
    i32 = jnp.int32
    ea = edge_attr.reshape(E)
    xu8 = jnp.pad(x.astype(jnp.uint8), (0, 4 * NXP - N))
    xp = lax.bitcast_convert_type(xu8.reshape(NXP, 4), i32)
    emb9 = jnp.zeros((10, 9), f32).at[:, :EMB].set(emb)
    embf = jnp.pad(emb9.reshape(90), (0, 6))

    parts = _sc_aggregate(ea, edge_index.astype(i32), xp, embf)
    bt = jnp.asarray(_BT)

    x_row = jnp.pad(x.astype(i32), (0, N_ACC - N),
                    constant_values=10).reshape(1, N_ACC)
    batch_row = jnp.pad(batch.astype(i32), (0, N_ACC - N),
                        constant_values=1023).reshape(1, N_ACC)

    stats = pl.pallas_call(
        _b1_body,
        grid=(GRID,),
        in_specs=[pl.BlockSpec((2, W, NB), lambda i: (0, 0, i)),
                  pl.BlockSpec((WOUT, W), lambda i: (0, 0))],
        out_specs=pl.BlockSpec((WOUT, 128), lambda i: (0, 0)),
        out_shape=jax.ShapeDtypeStruct((WOUT, 128), f32),
    )(parts, bt)

    bf = jnp.bfloat16
    w1p = jnp.zeros((200, WOUT), f32).at[:, :MSG].set(up_w1).astype(bf)
    gp = jnp.zeros((WOUT, 1), f32).at[:MSG, 0].set(up_bn_g)
    bp = jnp.zeros((WOUT, 1), f32).at[:MSG, 0].set(up_bn_b)
    embt = jnp.zeros((EMB, 16), f32).at[:, :10].set(emb.T)

    const = lambda shape: pl.BlockSpec(shape, lambda i: tuple(0 for _ in shape))
    xall, stats2 = pl.pallas_call(
        _b2_body,
        grid=(GRID,),
        in_specs=[
            pl.BlockSpec((2, W, NB), lambda i: (0, 0, i)),
            pl.BlockSpec((1, NB), lambda i: (0, i)),
            const((WOUT, W)),
            const((WOUT, 128)), const((WOUT, 1)), const((WOUT, 1)),
            const((200, WOUT)), const((200, 1)),
            const((200, 200)), const((200, 1)),
            const((200, 200)), const((200, 1)),
            const((EMB, 200)), const((EMB, 1)),
            const((EMB, 16)),
        ],
        out_specs=[
            pl.BlockSpec((2 * EMB, NB), lambda i: (0, i)),
            pl.BlockSpec((2 * EMB, 128), lambda i: (0, 0)),
        ],
        out_shape=[
            jax.ShapeDtypeStruct((2 * EMB, N_ACC), f32),
            jax.ShapeDtypeStruct((2 * EMB, 128), f32),
        ],
    )(parts, x_row, bt, stats, gp, bp,
      w1p, up_b1.reshape(200, 1), up_w2.astype(bf), up_b2.reshape(200, 1),
      up_w3.astype(bf), up_b3.reshape(200, 1), up_w4.astype(bf),
      up_b4.reshape(EMB, 1), embt)

    y32 = pl.pallas_call(
        _b3_body,
        grid=(GRID,),
        in_specs=[
            pl.BlockSpec((2 * EMB, NB), lambda i: (0, i)),
            pl.BlockSpec((1, NB), lambda i: (0, i)),
            const((2 * EMB, 128)), const((2 * EMB, 1)), const((2 * EMB, 1)),
            const((200, 2 * EMB)), const((200, 1)),
            const((200, 200)), const((200, 1)),
            const((200, 200)), const((200, 1)),
            const((1, 200)), const((1, 1)),
        ],
        out_specs=pl.BlockSpec((32, 32), lambda i: (0, 0)),
        out_shape=jax.ShapeDtypeStruct((32, 32), f32),
    )(xall, batch_row, stats2,
      ro_bn_g.reshape(2 * EMB, 1), ro_bn_b.reshape(2 * EMB, 1),
      ro_w1.astype(bf), ro_b1.reshape(200, 1), ro_w2.astype(bf),
      ro_b2.reshape(200, 1), ro_w3.astype(bf), ro_b3.reshape(200, 1),
      ro_w4.astype(bf), ro_b4.reshape(1, 1))

    return y32.reshape(1024, 1)[:NMOL]
